# baseline (speedup 1.0000x reference)
import functools
import jax
import jax.numpy as jnp
from jax import lax
from jax.experimental import pallas as pl
from jax.experimental.pallas import tpu as pltpu
from jax.experimental.pallas import tpu_sc as plsc

_L = 14
_H = 64
_S = 6
_FD = 256
_K = 17
_B = 592
_N = _B * _K
_E = 161024

_NP = 10240
_RB = 2048
_NG = _NP // _RB

_EC = 2
_ES = 16
_EK = 128
_EJ = 40
_EP = _EC * _ES * _EJ * _EK
_RPT = _NP // _ES
_ZR = 40
_NB = 4
_LA = 3


def _sc_edge_pass(uv, src4, dst4):
    mesh = plsc.VectorSubcoreMesh(core_axis_name="c", subcore_axis_name="s")

    @functools.partial(
        pl.kernel,
        out_type=jax.ShapeDtypeStruct((_EC, _NP, 128), jnp.bfloat16),
        mesh=mesh,
        compiler_params=pltpu.CompilerParams(use_tc_tiling_on_sc=False),
        scratch_types=[
            pltpu.VMEM((_EJ, _EK), jnp.int32),
            pltpu.VMEM((_EJ, _EK), jnp.int32),
        ] + [pltpu.VMEM((_EK, 128), jnp.bfloat16) for _ in range(_NB)] + [
            pltpu.VMEM((_ZR, 128), jnp.bfloat16),
            pltpu.VMEM_SHARED((_NP, 128), jnp.bfloat16),
            pltpu.VMEM_SHARED((_NP, 128), jnp.bfloat16),
        ] + [pltpu.SemaphoreType.DMA for _ in range(2 * _NB + 1)],
    )
    def k(uv_hbm, src_hbm, dst_hbm, out_hbm, src_v, dst_v, *rest):
        rows = list(rest[:_NB])
        zbuf = rest[_NB]
        acc = rest[_NB + 1]
        tbl = rest[_NB + 2]
        gsem = list(rest[_NB + 3:_NB + 3 + _NB])
        ssem = list(rest[_NB + 3 + _NB:_NB + 3 + 2 * _NB])
        tsem = rest[_NB + 3 + 2 * _NB]
        c = lax.axis_index("c")
        s = lax.axis_index("s")

        tcp = pltpu.async_copy(uv_hbm.at[pl.ds(s * _RPT, _RPT)],
                               tbl.at[pl.ds(s * _RPT, _RPT)], tsem)

        @pl.loop(0, _ZR)
        def _zrow(i):
            @pl.loop(0, 128, step=32)
            def _zcol(j):
                zbuf[i, pl.ds(j, 32)] = jnp.zeros((32,), jnp.bfloat16)

        icp1 = pltpu.async_copy(src_hbm.at[c].at[s], src_v, ssem[0])
        icp2 = pltpu.async_copy(dst_hbm.at[c].at[s], dst_v, ssem[1])

        @pl.loop(0, _RPT // _ZR)
        def _zcp(i):
            pltpu.async_copy(zbuf, acc.at[pl.ds(s * _RPT + i * _ZR, _ZR)],
                             gsem[0])

        @pl.loop(0, _RPT // _ZR)
        def _zwait(i):
            pltpu.make_async_copy(zbuf, acc.at[pl.ds(s * _RPT + i * _ZR, _ZR)],
                                  gsem[0]).wait()

        icp1.wait()
        icp2.wait()
        tcp.wait()

        plsc.subcore_barrier()

        def gstart(j, b):
            pltpu.async_copy(tbl.at[src_v.at[j]], rows[b], gsem[b])

        def gwait(j, b):
            pltpu.make_async_copy(tbl.at[src_v.at[j]], rows[b],
                                  gsem[b]).wait()

        def sstart(j, b):
            pltpu.async_copy(rows[b], acc.at[dst_v.at[j]], ssem[b],
                             add=True)

        def swait(j, b):
            pltpu.make_async_copy(rows[b], acc.at[dst_v.at[j]],
                                  ssem[b]).wait()

        for b in range(_LA):
            gstart(b, b)

        @pl.loop(0, _EJ, step=_NB)
        def _grp(j0):
            for b in range(_NB):
                j = j0 + b
                gwait(j, b)

                @pl.when(j >= 1)
                def _():
                    swait(j - 1, (b - 1) % _NB)
                sstart(j, b)

                @pl.when(j + _LA < _EJ)
                def _():
                    gstart(j + _LA, (b + _LA) % _NB)

        swait(_EJ - 1, (_EJ - 1) % _NB)

        plsc.subcore_barrier()

        pltpu.sync_copy(acc.at[pl.ds(s * _RPT, _RPT)],
                        out_hbm.at[c].at[pl.ds(s * _RPT, _RPT)])

    return k(uv, src4, dst4)


def _ln(v, g, b):
    mu = v.mean(-1, keepdims=True)
    var = ((v - mu) ** 2).mean(-1, keepdims=True)
    return (v - mu) * jax.lax.rsqrt(var + 1e-5) * g + b


def _store_uv(uv_ref, hh, tn):
    msg = hh + 1e-7
    v = jnp.exp(msg * tn)
    uv_ref[...] = jnp.concatenate([v * msg, v], axis=1).astype(jnp.bfloat16)


def _enc_body(x_ref, we_ref, be_ref, tn_ref, h_ref, uv_ref):
    x0 = x_ref[:, 0:1]
    x1 = x_ref[:, 1:2]
    h = x0 * we_ref[0:1, :] + x1 * we_ref[1:2, :] + be_ref[...]
    h_ref[...] = h
    _store_uv(uv_ref, jax.nn.relu(h), tn_ref[...])


def _enc(x_pad, W_enc, b_enc, t0):
    f = pl.pallas_call(
        _enc_body,
        grid=(_NG,),
        in_specs=[
            pl.BlockSpec((_RB, 4), lambda r: (r, 0)),
            pl.BlockSpec((2, _H), lambda r: (0, 0)),
            pl.BlockSpec((1, _H), lambda r: (0, 0)),
            pl.BlockSpec((1, 1), lambda r: (0, 0)),
        ],
        out_specs=[
            pl.BlockSpec((_RB, _H), lambda r: (r, 0)),
            pl.BlockSpec((_RB, 2 * _H), lambda r: (r, 0)),
        ],
        out_shape=[
            jax.ShapeDtypeStruct((_NP, _H), jnp.float32),
            jax.ShapeDtypeStruct((_NP, 2 * _H), jnp.bfloat16),
        ],
    )
    return f(x_pad, W_enc, b_enc.reshape(1, _H), t0.reshape(1, 1))


def _layer_body(xn_ref, out_ref, nd_ref, w1_ref, b1_ref, mw_ref,
                mb_ref, w2_ref, b2_ref, lnw_ref, lnb_ref, tn_ref,
                outn_ref, uv_ref, *, first, lnwc_ref=None, lnbc_ref=None):
    out_prev = out_ref[...]
    if first:
        xn = xn_ref[...]
    else:
        xn = jax.nn.relu(_ln(out_prev, lnwc_ref[...], lnbc_ref[...]))
    nd = nd_ref[...].astype(jnp.float32)
    ndt = nd[0] + nd[1]
    agg = ndt[:, :_H] / (ndt[:, _H:] + 1e-16)
    z = xn + agg
    y = jnp.dot(z, w1_ref[...], preferred_element_type=jnp.float32, precision=lax.Precision.HIGHEST) + b1_ref[...]
    y = jax.nn.relu(_ln(y, mw_ref[...], mb_ref[...]))
    y = jnp.dot(y, w2_ref[...], preferred_element_type=jnp.float32, precision=lax.Precision.HIGHEST) + b2_ref[...]
    out_new = y if first else out_prev + y
    outn_ref[...] = out_new
    hh = jax.nn.relu(_ln(out_new, lnw_ref[...], lnb_ref[...]))
    _store_uv(uv_ref, hh, tn_ref[...])


def _first_body(xn_ref, out_ref, nd_ref, w1_ref, b1_ref, mw_ref, mb_ref,
                w2_ref, b2_ref, lnw_ref, lnb_ref, tn_ref, outn_ref, uv_ref):
    _layer_body(xn_ref, out_ref, nd_ref, w1_ref, b1_ref, mw_ref, mb_ref,
                w2_ref, b2_ref, lnw_ref, lnb_ref, tn_ref, outn_ref, uv_ref,
                first=True)


def _rest_body(out_ref, nd_ref, w1_ref, b1_ref, mw_ref, mb_ref, w2_ref,
               b2_ref, lnwc_ref, lnbc_ref, lnw_ref, lnb_ref, tn_ref,
               outn_ref, uv_ref):
    _layer_body(None, out_ref, nd_ref, w1_ref, b1_ref, mw_ref, mb_ref,
                w2_ref, b2_ref, lnw_ref, lnb_ref, tn_ref, outn_ref, uv_ref,
                first=False, lnwc_ref=lnwc_ref, lnbc_ref=lnbc_ref)


_VEC = lambda: pl.BlockSpec((1, _H), lambda r: (0, 0))
_VEC2 = lambda: pl.BlockSpec((1, 2 * _H), lambda r: (0, 0))


def _tc_layer(xn, out_prev, nd, W1i, b1i, mwi, mbi, W2i, b2i, lnwc, lnbc,
              lnw_n, lnb_n, tn, first):
    base_specs = [
        pl.BlockSpec((2, _RB, 2 * _H), lambda r: (0, r, 0)),
        pl.BlockSpec((_H, 2 * _H), lambda r: (0, 0)),
        _VEC2(), _VEC2(), _VEC2(),
        pl.BlockSpec((2 * _H, _H), lambda r: (0, 0)),
        _VEC(),
    ]
    row = lambda: pl.BlockSpec((_RB, _H), lambda r: (r, 0))
    if first:
        body = _first_body
        in_specs = [row(), row()] + base_specs + [_VEC(), _VEC(),
                                                  pl.BlockSpec((1, 1), lambda r: (0, 0))]
        args = [xn, out_prev]
    else:
        body = _rest_body
        in_specs = [row()] + base_specs + [_VEC(), _VEC(), _VEC(), _VEC(),
                                           pl.BlockSpec((1, 1), lambda r: (0, 0))]
        args = [out_prev]
    f = pl.pallas_call(
        body,
        grid=(_NG,),
        in_specs=in_specs,
        out_specs=[
            pl.BlockSpec((_RB, _H), lambda r: (r, 0)),
            pl.BlockSpec((_RB, 2 * _H), lambda r: (r, 0)),
        ],
        out_shape=[
            jax.ShapeDtypeStruct((_NP, _H), jnp.float32),
            jax.ShapeDtypeStruct((_NP, 2 * _H), jnp.bfloat16),
        ],
    )
    args += [nd, W1i, b1i.reshape(1, -1), mwi.reshape(1, -1),
             mbi.reshape(1, -1), W2i, b2i.reshape(1, -1)]
    if not first:
        args += [lnwc.reshape(1, -1), lnbc.reshape(1, -1)]
    args += [lnw_n.reshape(1, -1), lnb_n.reshape(1, -1), tn.reshape(1, 1)]
    return f(*args)


def _head_body(xr_ref, lnw_ref, lnb_ref, wfc_ref, bfc_ref, g_ref, bb_ref,
               out_ref):
    xr = jax.nn.relu(_ln(xr_ref[...], lnw_ref[...], lnb_ref[...]))
    cols = [xr[:, i, :] for i in range(_K)]
    g3 = cols[0]
    for i in range(1, 5):
        g3 = jnp.maximum(g3, cols[i])
    g4 = cols[5]
    for i in range(6, 11):
        g4 = jnp.maximum(g4, cols[i])
    g2 = cols[11]
    for i in range(12, 17):
        g2 = jnp.maximum(g2, cols[i])
    g1 = jnp.maximum(g3, g4)
    g0 = jnp.maximum(g1, g2)
    xp = [g0, g1, g2, g3, g4, g2]
    scale = 1.0 / jnp.sqrt(1.0 + 1e-5)
    feats = []
    for i in range(_S):
        f = jnp.dot(xp[i], wfc_ref[i], preferred_element_type=jnp.float32, precision=lax.Precision.HIGHEST)
        f = (f + bfc_ref[i:i + 1, 0, :]) * scale * g_ref[i:i + 1, 0, :] \
            + bb_ref[i:i + 1, 0, :]
        feats.append(f)
    f = jnp.concatenate(feats, axis=1)
    nrm = jnp.sqrt(jnp.sum(f * f, axis=1, keepdims=True))
    out_ref[...] = f / (nrm + 1e-12)


def _head(xr, lnw0, lnb0, W_fc, b_fc, bn_g, bn_b):
    f = pl.pallas_call(
        _head_body,
        grid=(1,),
        in_specs=[
            pl.BlockSpec((_B, _K, _H), lambda r: (0, 0, 0)),
            pl.BlockSpec((1, 1, _H), lambda r: (0, 0, 0)),
            pl.BlockSpec((1, 1, _H), lambda r: (0, 0, 0)),
            pl.BlockSpec((_S, _H, _FD), lambda r: (0, 0, 0)),
            pl.BlockSpec((_S, 1, _FD), lambda r: (0, 0, 0)),
            pl.BlockSpec((_S, 1, _FD), lambda r: (0, 0, 0)),
            pl.BlockSpec((_S, 1, _FD), lambda r: (0, 0, 0)),
        ],
        out_specs=pl.BlockSpec((_B, _S * _FD), lambda r: (0, 0)),
        out_shape=jax.ShapeDtypeStruct((_B, _S * _FD), jnp.float32),
    )
    return f(xr, lnw0.reshape(1, 1, _H), lnb0.reshape(1, 1, _H), W_fc,
             b_fc.reshape(_S, 1, _FD), bn_g.reshape(_S, 1, _FD),
             bn_b.reshape(_S, 1, _FD))


def kernel(x, edge_index, W_enc, b_enc, W1, b1, mw, mb, W2, b2, t, ln_w,
           ln_b, W_fc, b_fc, bn_g, bn_b):
    x_pad = jnp.pad(x, ((0, _NP - _N), (0, 1)))
    pad_ids = jnp.full((_EP - _E,), _NP - 1, jnp.int32)
    src4 = jnp.concatenate([edge_index[0], pad_ids]).reshape(_EC, _ES, _EJ, _EK)
    dst4 = jnp.concatenate([edge_index[1], pad_ids]).reshape(_EC, _ES, _EJ, _EK)

    xn, uv = _enc(x_pad, W_enc, b_enc, t[0])
    out = xn
    for i in range(_L):
        nd = _sc_edge_pass(uv, src4, dst4)
        nxt = i + 1 if i + 1 < _L else 0
        out, uv = _tc_layer(
            xn, out, nd, W1[i], b1[i], mw[i], mb[i], W2[i], b2[i],
            ln_w[i], ln_b[i], ln_w[nxt], ln_b[nxt], t[nxt], first=(i == 0))

    xr = out[:_N].reshape(_B, _K, _H)
    return _head(xr, ln_w[0], ln_b[0], W_fc, b_fc, bn_g, bn_b)


# speedup vs baseline: 1.0997x; 1.0997x over previous
import functools
import jax
import jax.numpy as jnp
from jax import lax
from jax.experimental import pallas as pl
from jax.experimental.pallas import tpu as pltpu
from jax.experimental.pallas import tpu_sc as plsc

_L = 14
_H = 64
_S = 6
_FD = 256
_K = 17
_B = 592
_N = _B * _K
_E = 161024

_NP = 10240
_RB = 2048
_NG = _NP // _RB

_EC = 2
_ES = 16
_EK = 128
_EJ = 40
_EP = _EC * _ES * _EJ * _EK
_RPT = _NP // _ES
_ZR = 40
_NB = 4
_LA = 3


def _sc_edge_pass(uv, src4, dst4):
    mesh = plsc.VectorSubcoreMesh(core_axis_name="c", subcore_axis_name="s")

    @functools.partial(
        pl.kernel,
        out_type=jax.ShapeDtypeStruct((_EC, _NP, 128), jnp.bfloat16),
        mesh=mesh,
        compiler_params=pltpu.CompilerParams(use_tc_tiling_on_sc=False),
        scratch_types=[
            pltpu.VMEM((_EJ, _EK), jnp.int32),
            pltpu.VMEM((_EJ, _EK), jnp.int32),
        ] + [pltpu.VMEM((_EK, 128), jnp.bfloat16) for _ in range(_NB)] + [
            pltpu.VMEM((_ZR, 128), jnp.bfloat16),
            pltpu.VMEM_SHARED((_NP, 128), jnp.bfloat16),
            pltpu.VMEM_SHARED((_NP, 128), jnp.bfloat16),
        ] + [pltpu.SemaphoreType.DMA for _ in range(2 * _NB + 1)],
    )
    def k(uv_hbm, src_hbm, dst_hbm, out_hbm, src_v, dst_v, *rest):
        rows = list(rest[:_NB])
        zbuf = rest[_NB]
        acc = rest[_NB + 1]
        tbl = rest[_NB + 2]
        gsem = list(rest[_NB + 3:_NB + 3 + _NB])
        ssem = list(rest[_NB + 3 + _NB:_NB + 3 + 2 * _NB])
        tsem = rest[_NB + 3 + 2 * _NB]
        c = lax.axis_index("c")
        s = lax.axis_index("s")

        tcp = pltpu.async_copy(uv_hbm.at[pl.ds(s * _RPT, _RPT)],
                               tbl.at[pl.ds(s * _RPT, _RPT)], tsem)

        @pl.loop(0, _ZR)
        def _zrow(i):
            @pl.loop(0, 128, step=32)
            def _zcol(j):
                zbuf[i, pl.ds(j, 32)] = jnp.zeros((32,), jnp.bfloat16)

        icp1 = pltpu.async_copy(src_hbm.at[c].at[s], src_v, ssem[0])
        icp2 = pltpu.async_copy(dst_hbm.at[c].at[s], dst_v, ssem[1])

        @pl.loop(0, _RPT // _ZR)
        def _zcp(i):
            pltpu.async_copy(zbuf, acc.at[pl.ds(s * _RPT + i * _ZR, _ZR)],
                             gsem[0])

        @pl.loop(0, _RPT // _ZR)
        def _zwait(i):
            pltpu.make_async_copy(zbuf, acc.at[pl.ds(s * _RPT + i * _ZR, _ZR)],
                                  gsem[0]).wait()

        icp1.wait()
        icp2.wait()
        tcp.wait()

        plsc.subcore_barrier()

        def gstart(j, b):
            pltpu.async_copy(tbl.at[src_v.at[j]], rows[b], gsem[b])

        def gwait(j, b):
            pltpu.make_async_copy(tbl.at[src_v.at[j]], rows[b],
                                  gsem[b]).wait()

        def sstart(j, b):
            pltpu.async_copy(rows[b], acc.at[dst_v.at[j]], ssem[b],
                             add=True)

        def swait(j, b):
            pltpu.make_async_copy(rows[b], acc.at[dst_v.at[j]],
                                  ssem[b]).wait()

        for b in range(_LA):
            gstart(b, b)

        @pl.loop(0, _EJ, step=_NB)
        def _grp(j0):
            for b in range(_NB):
                j = j0 + b
                gwait(j, b)

                @pl.when(j >= 1)
                def _():
                    swait(j - 1, (b - 1) % _NB)
                sstart(j, b)

                @pl.when(j + _LA < _EJ)
                def _():
                    gstart(j + _LA, (b + _LA) % _NB)

        swait(_EJ - 1, (_EJ - 1) % _NB)

        plsc.subcore_barrier()

        pltpu.sync_copy(acc.at[pl.ds(s * _RPT, _RPT)],
                        out_hbm.at[c].at[pl.ds(s * _RPT, _RPT)])

    return k(uv, src4, dst4)


def _ln(v, g, b):
    mu = v.mean(-1, keepdims=True)
    var = ((v - mu) ** 2).mean(-1, keepdims=True)
    return (v - mu) * jax.lax.rsqrt(var + 1e-5) * g + b


def _store_uv(uv_ref, hh, tn):
    msg = hh + 1e-7
    v = jnp.exp(msg * tn)
    uv_ref[...] = jnp.concatenate([v * msg, v], axis=1).astype(jnp.bfloat16)


def _enc_body(x_ref, we_ref, be_ref, tn_ref, h_ref, uv_ref):
    x0 = x_ref[:, 0:1]
    x1 = x_ref[:, 1:2]
    h = x0 * we_ref[0:1, :] + x1 * we_ref[1:2, :] + be_ref[...]
    h_ref[...] = h
    _store_uv(uv_ref, jax.nn.relu(h), tn_ref[...])


def _enc(x_pad, W_enc, b_enc, t0):
    f = pl.pallas_call(
        _enc_body,
        grid=(_NG,),
        in_specs=[
            pl.BlockSpec((_RB, 4), lambda r: (r, 0)),
            pl.BlockSpec((2, _H), lambda r: (0, 0)),
            pl.BlockSpec((1, _H), lambda r: (0, 0)),
            pl.BlockSpec((1, 1), lambda r: (0, 0)),
        ],
        out_specs=[
            pl.BlockSpec((_RB, _H), lambda r: (r, 0)),
            pl.BlockSpec((_RB, 2 * _H), lambda r: (r, 0)),
        ],
        out_shape=[
            jax.ShapeDtypeStruct((_NP, _H), jnp.float32),
            jax.ShapeDtypeStruct((_NP, 2 * _H), jnp.bfloat16),
        ],
    )
    return f(x_pad, W_enc, b_enc.reshape(1, _H), t0.reshape(1, 1))


def _layer_body(first, xn_ref, out_ref, nd_ref, w1_ref, b1_ref, mw_ref,
                mb_ref, w2_ref, b2_ref, lnw_ref, lnb_ref, tn_ref,
                outn_ref, hh_ref, uv_ref):
    nd = nd_ref[...].astype(jnp.float32)
    ndt = nd[0] + nd[1]
    agg = ndt[:, :_H] / (ndt[:, _H:] + 1e-16)
    z = xn_ref[...] + agg
    y = jnp.dot(z, w1_ref[...], preferred_element_type=jnp.float32, precision=lax.Precision.HIGHEST) + b1_ref[...]
    y = jax.nn.relu(_ln(y, mw_ref[...], mb_ref[...]))
    y = jnp.dot(y, w2_ref[...], preferred_element_type=jnp.float32, precision=lax.Precision.HIGHEST) + b2_ref[...]
    out_new = y if first else out_ref[...] + y
    outn_ref[...] = out_new
    hh = jax.nn.relu(_ln(out_new, lnw_ref[...], lnb_ref[...]))
    hh_ref[...] = hh
    _store_uv(uv_ref, hh, tn_ref[...])


def _tc_layer(xn, out_prev, nd, W1i, b1i, mwi, mbi, W2i, b2i, lnw_n, lnb_n,
              tn, first):
    f = pl.pallas_call(
        functools.partial(_layer_body, first),
        grid=(_NG,),
        in_specs=[
            pl.BlockSpec((_RB, _H), lambda r: (r, 0)),
            pl.BlockSpec((_RB, _H), lambda r: (r, 0)),
            pl.BlockSpec((2, _RB, 2 * _H), lambda r: (0, r, 0)),
            pl.BlockSpec((_H, 2 * _H), lambda r: (0, 0)),
            pl.BlockSpec((1, 2 * _H), lambda r: (0, 0)),
            pl.BlockSpec((1, 2 * _H), lambda r: (0, 0)),
            pl.BlockSpec((1, 2 * _H), lambda r: (0, 0)),
            pl.BlockSpec((2 * _H, _H), lambda r: (0, 0)),
            pl.BlockSpec((1, _H), lambda r: (0, 0)),
            pl.BlockSpec((1, _H), lambda r: (0, 0)),
            pl.BlockSpec((1, _H), lambda r: (0, 0)),
            pl.BlockSpec((1, 1), lambda r: (0, 0)),
        ],
        out_specs=[
            pl.BlockSpec((_RB, _H), lambda r: (r, 0)),
            pl.BlockSpec((_RB, _H), lambda r: (r, 0)),
            pl.BlockSpec((_RB, 2 * _H), lambda r: (r, 0)),
        ],
        out_shape=[
            jax.ShapeDtypeStruct((_NP, _H), jnp.float32),
            jax.ShapeDtypeStruct((_NP, _H), jnp.float32),
            jax.ShapeDtypeStruct((_NP, 2 * _H), jnp.bfloat16),
        ],
    )
    return f(xn, out_prev, nd, W1i, b1i.reshape(1, -1), mwi.reshape(1, -1),
             mbi.reshape(1, -1), W2i, b2i.reshape(1, -1),
             lnw_n.reshape(1, -1), lnb_n.reshape(1, -1), tn.reshape(1, 1))


def _head_body(xr_ref, wfc_ref, bfc_ref, g_ref, bb_ref, out_ref):
    xr = xr_ref[...]
    cols = [xr[:, i, :] for i in range(_K)]
    g3 = cols[0]
    for i in range(1, 5):
        g3 = jnp.maximum(g3, cols[i])
    g4 = cols[5]
    for i in range(6, 11):
        g4 = jnp.maximum(g4, cols[i])
    g2 = cols[11]
    for i in range(12, 17):
        g2 = jnp.maximum(g2, cols[i])
    g1 = jnp.maximum(g3, g4)
    g0 = jnp.maximum(g1, g2)
    xp = [g0, g1, g2, g3, g4, g2]
    scale = 1.0 / jnp.sqrt(1.0 + 1e-5)
    feats = []
    for i in range(_S):
        f = jnp.dot(xp[i], wfc_ref[i], preferred_element_type=jnp.float32, precision=lax.Precision.HIGHEST)
        f = (f + bfc_ref[i:i + 1, 0, :]) * scale * g_ref[i:i + 1, 0, :] \
            + bb_ref[i:i + 1, 0, :]
        feats.append(f)
    f = jnp.concatenate(feats, axis=1)
    nrm = jnp.sqrt(jnp.sum(f * f, axis=1, keepdims=True))
    out_ref[...] = f / (nrm + 1e-12)


def _head(xr, W_fc, b_fc, bn_g, bn_b):
    f = pl.pallas_call(
        _head_body,
        grid=(1,),
        in_specs=[
            pl.BlockSpec((_B, _K, _H), lambda r: (0, 0, 0)),
            pl.BlockSpec((_S, _H, _FD), lambda r: (0, 0, 0)),
            pl.BlockSpec((_S, 1, _FD), lambda r: (0, 0, 0)),
            pl.BlockSpec((_S, 1, _FD), lambda r: (0, 0, 0)),
            pl.BlockSpec((_S, 1, _FD), lambda r: (0, 0, 0)),
        ],
        out_specs=pl.BlockSpec((_B, _S * _FD), lambda r: (0, 0)),
        out_shape=jax.ShapeDtypeStruct((_B, _S * _FD), jnp.float32),
    )
    return f(xr, W_fc, b_fc.reshape(_S, 1, _FD), bn_g.reshape(_S, 1, _FD),
             bn_b.reshape(_S, 1, _FD))


def kernel(x, edge_index, W_enc, b_enc, W1, b1, mw, mb, W2, b2, t, ln_w,
           ln_b, W_fc, b_fc, bn_g, bn_b):
    x_pad = jnp.pad(x, ((0, _NP - _N), (0, 1)))
    pad_ids = jnp.full((_EP - _E,), _NP - 1, jnp.int32)
    src4 = jnp.concatenate([edge_index[0], pad_ids]).reshape(_EC, _ES, _EJ, _EK)
    dst4 = jnp.concatenate([edge_index[1], pad_ids]).reshape(_EC, _ES, _EJ, _EK)

    xn, uv = _enc(x_pad, W_enc, b_enc, t[0])
    out = xn
    for i in range(_L):
        nd = _sc_edge_pass(uv, src4, dst4)
        nxt = i + 1 if i + 1 < _L else 0
        out, xn, uv = _tc_layer(
            xn, out, nd, W1[i], b1[i], mw[i], mb[i], W2[i], b2[i],
            ln_w[nxt], ln_b[nxt], t[nxt], first=(i == 0))

    xr = xn[:_N].reshape(_B, _K, _H)
    return _head(xr, W_fc, b_fc, bn_g, bn_b)
